# baseline (speedup 1.0000x reference)
import functools

import jax
import jax.numpy as jnp
from jax.experimental import pallas as pl
from jax.experimental.pallas import tpu as pltpu

_NEG_SLOPE = 0.1
_B = 16
_T = 64
_L_OUTS = (505, 247, 121, 59)
_KS = (15, 12, 7, 4)


def _leaky(h):
    return jnp.where(h > 0, h, _NEG_SLOPE * h)


def _conv_layer(read_in, write_out, w_ref, b_ref, l_out, k_size, activate):
    n_chunks = (l_out + _T - 1) // _T
    bias = b_ref[0][None, None, :]
    for c in range(n_chunks):
        s = c * _T
        acc = None
        for k in range(k_size):
            lhs = read_in(2 * s + k, _T, 2)
            lhs2 = lhs.reshape(_B * _T, 128).astype(jnp.bfloat16)
            d = jax.lax.dot_general(
                lhs2, w_ref[k],
                (((1,), (0,)), ((), ())),
                preferred_element_type=jnp.float32)
            acc = d if acc is None else acc + d
        acc = acc.reshape(_B, _T, 128) + bias
        if activate:
            acc = _leaky(acc)
        write_out(s, acc)


def _dec_kernel(x_ref, w1_ref, b1_ref, w2_ref, b2_ref, w3_ref, b3_ref,
                w4_ref, b4_ref, ct_ref, c2_ref, o_ref,
                xt_ref, h1_ref, h2_ref, h3_ref):
    for b in range(_B):
        xt_ref[b, 0:1024, :] = jnp.transpose(x_ref[b], (1, 0)).astype(jnp.float32)
    xt_ref[:, 1024:1040, :] = jnp.zeros((_B, 16, 128), jnp.float32)
    _conv_layer(
        lambda s, t, st: xt_ref[:, pl.ds(s, t, st), :],
        lambda s, v: h1_ref.__setitem__(
            (slice(None), pl.ds(s, _T), slice(None)), v),
        w1_ref, b1_ref, _L_OUTS[0], _KS[0], True)
    _conv_layer(
        lambda s, t, st: h1_ref[:, pl.ds(s, t, st), :],
        lambda s, v: h2_ref.__setitem__(
            (slice(None), pl.ds(s, _T), slice(None)), v),
        w2_ref, b2_ref, _L_OUTS[1], _KS[1], True)
    _conv_layer(
        lambda s, t, st: h2_ref[:, pl.ds(s, t, st), :],
        lambda s, v: h3_ref.__setitem__(
            (slice(None), pl.ds(s, _T), slice(None)), v),
        w3_ref, b3_ref, _L_OUTS[2], _KS[2], True)
    h4 = None
    for k in range(_KS[3]):
        lhs = h3_ref[:, pl.ds(k, _T, 2), :].reshape(_B * _T, 128).astype(jnp.bfloat16)
        d = jax.lax.dot_general(
            lhs, w4_ref[k], (((1,), (0,)), ((), ())),
            preferred_element_type=jnp.float32)
        h4 = d if h4 is None else h4 + d
    h4 = h4.reshape(_B, _T, 128) + b4_ref[0][None, None, :]

    zc = None
    for l in range(_L_OUTS[3]):
        hl = h4[:, l, :]
        d = jax.lax.dot_general(
            hl, ct_ref[l], (((1,), (0,)), ((), ())),
            preferred_element_type=jnp.float32)
        zc = d if zc is None else zc + d
    hv = h4[:, 0:_L_OUTS[3], :]
    z2 = jnp.sum(hv * hv, axis=(1, 2))[:, None]
    d2 = z2 + c2_ref[...] - 2.0 * zc
    q = 1.0 / (1.0 + d2)
    o_ref[...] = q / jnp.sum(q, axis=1, keepdims=True)


def _c2_kernel(ct_ref, o_ref):
    c = ct_ref[...]
    o_ref[...] = jnp.sum(c * c, axis=(0, 1), keepdims=False)[None, :]


@jax.jit
def kernel(x, w1, b1, w2, b2, w3, b3, w4, b4, centers):
    n, c_ch, l_in = x.shape
    xb = x.astype(jnp.bfloat16)
    wt1 = jnp.transpose(w1, (2, 1, 0)).astype(jnp.bfloat16)
    wt2 = jnp.transpose(w2, (2, 1, 0)).astype(jnp.bfloat16)
    wt3 = jnp.transpose(w3, (2, 1, 0)).astype(jnp.bfloat16)
    wt4 = jnp.transpose(w4, (2, 1, 0)).astype(jnp.bfloat16)
    ct = jnp.transpose(centers.reshape(64, c_ch, _L_OUTS[3]), (2, 1, 0))
    b1r, b2r, b3r, b4r = (b.reshape(1, c_ch) for b in (b1, b2, b3, b4))

    c2 = pl.pallas_call(
        _c2_kernel,
        out_shape=jax.ShapeDtypeStruct((1, 64), jnp.float32),
    )(ct)

    grid = (n // _B,)
    full = lambda shape: pl.BlockSpec(shape, lambda i: (0,) * len(shape))
    out = pl.pallas_call(
        _dec_kernel,
        grid=grid,
        in_specs=[
            pl.BlockSpec((_B, c_ch, l_in), lambda i: (i, 0, 0)),
            full((_KS[0], c_ch, c_ch)), full((1, c_ch)),
            full((_KS[1], c_ch, c_ch)), full((1, c_ch)),
            full((_KS[2], c_ch, c_ch)), full((1, c_ch)),
            full((_KS[3], c_ch, c_ch)), full((1, c_ch)),
            full((_L_OUTS[3], c_ch, 64)), full((1, 64)),
        ],
        out_specs=pl.BlockSpec((_B, 64), lambda i: (i, 0)),
        out_shape=jax.ShapeDtypeStruct((n, 64), jnp.float32),
        scratch_shapes=[
            pltpu.VMEM((_B, 1040, c_ch), jnp.float32),
            pltpu.VMEM((_B, 528, c_ch), jnp.float32),
            pltpu.VMEM((_B, 264, c_ch), jnp.float32),
            pltpu.VMEM((_B, 136, c_ch), jnp.float32),
        ],
        compiler_params=pltpu.CompilerParams(
            dimension_semantics=("parallel",),
            vmem_limit_bytes=100 * 1024 * 1024,
        ),
    )(xb, wt1, b1r, wt2, b2r, wt3, b3r, wt4, b4r, ct, c2)
    return out


# speedup vs baseline: 1.2345x; 1.2345x over previous
import functools

import jax
import jax.numpy as jnp
from jax.experimental import pallas as pl
from jax.experimental.pallas import tpu as pltpu

_NEG_SLOPE = 0.1
_B = 16
_T = 64
_L_OUTS = (505, 247, 121, 59)
_KS = (15, 12, 7, 4)


def _leaky(h):
    return jnp.where(h > 0, h, _NEG_SLOPE * h)


def _conv_layer(read_in, write_out, w_ref, b_ref, l_out, k_size, activate):
    n_chunks = (l_out + _T - 1) // _T
    bias = b_ref[0][None, None, :]
    for c in range(n_chunks):
        s = c * _T
        acc = None
        for k in range(k_size):
            lhs = read_in(2 * s + k, _T, 2)
            lhs2 = lhs.reshape(_B * _T, 128).astype(jnp.bfloat16)
            d = jax.lax.dot_general(
                lhs2, w_ref[k],
                (((1,), (0,)), ((), ())),
                preferred_element_type=jnp.float32)
            acc = d if acc is None else acc + d
        acc = acc.reshape(_B, _T, 128) + bias
        if activate:
            acc = _leaky(acc)
        write_out(s, acc)


def _dec_kernel(x_ref, w1_ref, b1_ref, w2_ref, b2_ref, w3_ref, b3_ref,
                w4_ref, b4_ref, ct_ref, c2_ref, o_ref,
                xt_ref, h1_ref, h2_ref, h3_ref):
    for b in range(_B):
        xt_ref[b, 0:1024, :] = jnp.transpose(x_ref[b], (1, 0))
    xt_ref[:, 1024:1040, :] = jnp.zeros((_B, 16, 128), jnp.float32)
    _conv_layer(
        lambda s, t, st: xt_ref[:, pl.ds(s, t, st), :],
        lambda s, v: h1_ref.__setitem__(
            (slice(None), pl.ds(s, _T), slice(None)), v),
        w1_ref, b1_ref, _L_OUTS[0], _KS[0], True)
    _conv_layer(
        lambda s, t, st: h1_ref[:, pl.ds(s, t, st), :],
        lambda s, v: h2_ref.__setitem__(
            (slice(None), pl.ds(s, _T), slice(None)), v),
        w2_ref, b2_ref, _L_OUTS[1], _KS[1], True)
    _conv_layer(
        lambda s, t, st: h2_ref[:, pl.ds(s, t, st), :],
        lambda s, v: h3_ref.__setitem__(
            (slice(None), pl.ds(s, _T), slice(None)), v),
        w3_ref, b3_ref, _L_OUTS[2], _KS[2], True)
    h4 = None
    for k in range(_KS[3]):
        lhs = h3_ref[:, pl.ds(k, _T, 2), :].reshape(_B * _T, 128).astype(jnp.bfloat16)
        d = jax.lax.dot_general(
            lhs, w4_ref[k], (((1,), (0,)), ((), ())),
            preferred_element_type=jnp.float32)
        h4 = d if h4 is None else h4 + d
    h4 = h4.reshape(_B, _T, 128) + b4_ref[0][None, None, :]

    zc = None
    for l in range(_L_OUTS[3]):
        hl = h4[:, l, :]
        d = jax.lax.dot_general(
            hl, ct_ref[l], (((1,), (0,)), ((), ())),
            preferred_element_type=jnp.float32)
        zc = d if zc is None else zc + d
    hv = h4[:, 0:_L_OUTS[3], :]
    z2 = jnp.sum(hv * hv, axis=(1, 2))[:, None]
    d2 = z2 + c2_ref[...] - 2.0 * zc
    q = 1.0 / (1.0 + d2)
    o_ref[...] = q / jnp.sum(q, axis=1, keepdims=True)


def _c2_kernel(ct_ref, o_ref):
    c = ct_ref[...]
    o_ref[...] = jnp.sum(c * c, axis=(0, 1), keepdims=False)[None, :]


@jax.jit
def kernel(x, w1, b1, w2, b2, w3, b3, w4, b4, centers):
    n, c_ch, l_in = x.shape
    wt1 = jnp.transpose(w1, (2, 1, 0)).astype(jnp.bfloat16)
    wt2 = jnp.transpose(w2, (2, 1, 0)).astype(jnp.bfloat16)
    wt3 = jnp.transpose(w3, (2, 1, 0)).astype(jnp.bfloat16)
    wt4 = jnp.transpose(w4, (2, 1, 0)).astype(jnp.bfloat16)
    ct = jnp.transpose(centers.reshape(64, c_ch, _L_OUTS[3]), (2, 1, 0))
    b1r, b2r, b3r, b4r = (b.reshape(1, c_ch) for b in (b1, b2, b3, b4))

    c2 = pl.pallas_call(
        _c2_kernel,
        out_shape=jax.ShapeDtypeStruct((1, 64), jnp.float32),
    )(ct)

    grid = (n // _B,)
    full = lambda shape: pl.BlockSpec(shape, lambda i: (0,) * len(shape))
    out = pl.pallas_call(
        _dec_kernel,
        grid=grid,
        in_specs=[
            pl.BlockSpec((_B, c_ch, l_in), lambda i: (i, 0, 0)),
            full((_KS[0], c_ch, c_ch)), full((1, c_ch)),
            full((_KS[1], c_ch, c_ch)), full((1, c_ch)),
            full((_KS[2], c_ch, c_ch)), full((1, c_ch)),
            full((_KS[3], c_ch, c_ch)), full((1, c_ch)),
            full((_L_OUTS[3], c_ch, 64)), full((1, 64)),
        ],
        out_specs=pl.BlockSpec((_B, 64), lambda i: (i, 0)),
        out_shape=jax.ShapeDtypeStruct((n, 64), jnp.float32),
        scratch_shapes=[
            pltpu.VMEM((_B, 1040, c_ch), jnp.float32),
            pltpu.VMEM((_B, 528, c_ch), jnp.float32),
            pltpu.VMEM((_B, 264, c_ch), jnp.float32),
            pltpu.VMEM((_B, 136, c_ch), jnp.float32),
        ],
        compiler_params=pltpu.CompilerParams(
            dimension_semantics=("parallel",),
            vmem_limit_bytes=100 * 1024 * 1024,
        ),
    )(x, wt1, b1r, wt2, b2r, wt3, b3r, wt4, b4r, ct, c2)
    return out
